# baseline (speedup 1.0000x reference)
import functools

import jax
import jax.numpy as jnp
from jax import lax
from jax.experimental import pallas as pl
from jax.experimental.pallas import tpu as pltpu
from jax.experimental.pallas import tpu_sc as plsc

_B = 4096
_L = 200
_D = 128
_FC = 2048
_NC = 2
_NS = 16
_NW = _NC * _NS
_BPW = _B // _NW
_CHUNKS = ((0, 72), (72, 48), (120, 40), (160, 40))
_DV = _D // 16
_FSTG = 8


def _accumulate(rows_ref, lenv, n):
    init = tuple(jnp.zeros((16,), jnp.float32) for _ in range(_DV))
    nfull = n // 8

    def body(j, carry):
        accs = list(carry)
        for r in range(8):
            for c in range(_DV):
                accs[c] = accs[c] + rows_ref[8 * j + r, pl.ds(16 * c, 16)]
        return tuple(accs)

    t0 = nfull * 8
    acc = list(lax.fori_loop(0, nfull, body, init))
    for r in range(8):
        m = jnp.broadcast_to(t0 + r, (16,)) < lenv
        for c in range(_DV):
            acc[c] = acc[c] + jnp.where(
                m, rows_ref[t0 + r, pl.ds(16 * c, 16)], 0.0
            )
    inv = 1.0 / jnp.maximum(lenv, 1).astype(jnp.float32)
    return tuple(acc[c] * inv for c in range(_DV))


def _encoder_kernel(htok_hbm, ptok_hbm, lens_hbm, table_hbm,
                    feat_hbm,
                    toks, lens_v, hrows, prows, fstage,
                    sem_t, sem_h, sem_p):
    wid = lax.axis_index("s") * _NC + lax.axis_index("c")
    base = pl.multiple_of(wid * _BPW, _BPW)

    pltpu.sync_copy(lens_hbm.at[pl.ds(base, _BPW)], lens_v)

    def tok_src(hbm, b):
        off = pl.multiple_of((base + b) * _L, 8)
        return hbm.at[pl.ds(off, _L)]

    def tok_slot(par, e):
        return pl.multiple_of(par * 512 + e * 256, 8)

    for b0 in range(2):
        pltpu.sync_copy(
            tok_src(htok_hbm, b0), toks.at[pl.ds(tok_slot(b0, 0), _L)]
        )
        pltpu.sync_copy(
            tok_src(ptok_hbm, b0), toks.at[pl.ds(tok_slot(b0, 1), _L)]
        )

    def issue(e, rows, par, sem, n):
        for off, sz in _CHUNKS:

            def one(off=off, sz=sz):
                pltpu.async_copy(
                    table_hbm.at[toks.at[pl.ds(tok_slot(par, e) + off, sz)]],
                    rows.at[par, pl.ds(off, sz)],
                    sem,
                )

            if off == 0:
                one()
            else:
                pl.when(n > off)(one)

    def drain(rows, sem, n):
        for off, sz in _CHUNKS:

            def one(off=off, sz=sz):
                pltpu.make_async_copy(
                    table_hbm.at[pl.ds(0, sz)], rows.at[0, pl.ds(off, sz)],
                    sem,
                ).wait()

            if off == 0:
                one()
            else:
                pl.when(n > off)(one)

    def drain_tok():
        pltpu.make_async_copy(
            htok_hbm.at[pl.ds(0, _L)], toks.at[pl.ds(0, _L)], sem_t
        ).wait()
        pltpu.make_async_copy(
            htok_hbm.at[pl.ds(0, _L)], toks.at[pl.ds(0, _L)], sem_t
        ).wait()

    def hlenv(b):
        return lens_v[b, pl.ds(0, 16)]

    def plenv(b):
        return lens_v[b, pl.ds(16, 16)]

    def hlen(b):
        return lens_v[b, pl.ds(0, 16)][0]

    def plen(b):
        return lens_v[b, pl.ds(16, 16)][0]

    issue(0, hrows, 0, sem_h, hlen(0))
    issue(1, prows, 0, sem_p, plen(0))

    def item(b, carry):
        par = lax.rem(b, 2)
        nxt = 1 - par

        drain(hrows, sem_h, hlen(b))
        drain(prows, sem_p, plen(b))

        @pl.when(b < _BPW - 2)
        def _():
            pltpu.async_copy(
                tok_src(htok_hbm, b + 2),
                toks.at[pl.ds(tok_slot(par, 0), _L)], sem_t,
            )
            pltpu.async_copy(
                tok_src(ptok_hbm, b + 2),
                toks.at[pl.ds(tok_slot(par, 1), _L)], sem_t,
            )

        @pl.when(jnp.logical_and(b > 0, b < _BPW - 1))
        def _():
            drain_tok()

        @pl.when(b < _BPW - 1)
        def _():
            issue(0, hrows, nxt, sem_h, hlen(b + 1))
            issue(1, prows, nxt, sem_p, plen(b + 1))

        hh = _accumulate(hrows.at[par], hlenv(b), hlen(b))
        hp = _accumulate(prows.at[par], plenv(b), plen(b))

        row = lax.rem(b, _FSTG)
        for c in range(_DV):
            p = hp[c]
            h = hh[c]
            fstage[row, pl.ds(16 * c, 16)] = p
            fstage[row, pl.ds(_D + 16 * c, 16)] = h
            fstage[row, pl.ds(2 * _D + 16 * c, 16)] = jnp.abs(p - h)
            fstage[row, pl.ds(3 * _D + 16 * c, 16)] = p * h

        @pl.when(row == _FSTG - 1)
        def _():
            off = pl.multiple_of(base + b - (_FSTG - 1), _FSTG)
            pltpu.sync_copy(fstage, feat_hbm.at[pl.ds(off, _FSTG)])

        return carry

    lax.fori_loop(0, _BPW, item, 0)


@jax.jit
def _encode_sc(htok, ptok, lens, table):
    mesh = plsc.VectorSubcoreMesh(core_axis_name="c", subcore_axis_name="s")
    k = functools.partial(
        pl.kernel,
        mesh=mesh,
        out_type=jax.ShapeDtypeStruct((_B, 4 * _D), jnp.float32),
        scratch_types=[
            pltpu.VMEM((1024,), jnp.int32),
            pltpu.VMEM((_BPW, 32), jnp.int32),
            pltpu.VMEM((2, _L + 8, _D), jnp.float32),
            pltpu.VMEM((2, _L + 8, _D), jnp.float32),
            pltpu.VMEM((_FSTG, 4 * _D), jnp.float32),
            pltpu.SemaphoreType.DMA,
            pltpu.SemaphoreType.DMA,
            pltpu.SemaphoreType.DMA,
        ],
    )(_encoder_kernel)
    return k(htok, ptok, lens, table)


def _tc_body(W1_ref, W2_ref, W3p_ref, b1_ref, b2_ref, b3p_ref, f_ref,
             out_ref, Wcp_scr, bcp_scr):
    hi = jax.lax.Precision.HIGHEST

    @pl.when(pl.program_id(0) == 0)
    def _():
        W23 = lax.dot_general(
            W2_ref[...], W3p_ref[...], (((1,), (0,)), ((), ())),
            preferred_element_type=jnp.float32, precision=hi,
        )
        Wcp_scr[...] = lax.dot_general(
            W1_ref[...], W23, (((1,), (0,)), ((), ())),
            preferred_element_type=jnp.float32, precision=hi,
        )
        bc1 = jnp.sum(W23 * b1_ref[...], axis=0, keepdims=True)
        bc2 = jnp.sum(W3p_ref[...] * b2_ref[...], axis=0, keepdims=True)
        bcp_scr[...] = bc1 + bc2 + b3p_ref[...]

    out_ref[...] = lax.dot_general(
        f_ref[...], Wcp_scr[...], (((1,), (0,)), ((), ())),
        preferred_element_type=jnp.float32, precision=hi,
    ) + bcp_scr[...]


def kernel(hypothesis_tokens, hypothesis_len, premise_tokens, premise_len,
           emb_table, W1, b1, W2, b2, W3, b3):
    htok = hypothesis_tokens.astype(jnp.int32).reshape(_B * _L)
    ptok = premise_tokens.astype(jnp.int32).reshape(_B * _L)
    table = emb_table.astype(jnp.float32)
    lens = jnp.concatenate(
        [
            jnp.broadcast_to(hypothesis_len.astype(jnp.int32)[:, None], (_B, 16)),
            jnp.broadcast_to(premise_len.astype(jnp.int32)[:, None], (_B, 16)),
        ],
        axis=1,
    )
    feats = _encode_sc(htok, ptok, lens, table)

    out_dim = W3.shape[1]
    W3p = jnp.pad(W3, ((0, 0), (0, 128 - out_dim)))
    b3p = jnp.pad(b3, (0, 128 - out_dim)).reshape(1, 128)

    bm = 512
    outp = pl.pallas_call(
        _tc_body,
        grid=(_B // bm,),
        in_specs=[
            pl.BlockSpec((4 * _D, _FC), lambda i: (0, 0)),
            pl.BlockSpec((_FC, _FC), lambda i: (0, 0)),
            pl.BlockSpec((_FC, 128), lambda i: (0, 0)),
            pl.BlockSpec((_FC, 1), lambda i: (0, 0)),
            pl.BlockSpec((_FC, 1), lambda i: (0, 0)),
            pl.BlockSpec((1, 128), lambda i: (0, 0)),
            pl.BlockSpec((bm, 4 * _D), lambda i: (i, 0)),
        ],
        out_specs=pl.BlockSpec((bm, 128), lambda i: (i, 0)),
        out_shape=jax.ShapeDtypeStruct((_B, 128), jnp.float32),
        scratch_shapes=[
            pltpu.VMEM((4 * _D, 128), jnp.float32),
            pltpu.VMEM((1, 128), jnp.float32),
        ],
    )(W1, W2, W3p, b1.reshape(_FC, 1), b2.reshape(_FC, 1), b3p, feats)

    return outp[:, :out_dim]


# speedup vs baseline: 1.1202x; 1.1202x over previous
import functools

import jax
import jax.numpy as jnp
from jax import lax
from jax.experimental import pallas as pl
from jax.experimental.pallas import tpu as pltpu
from jax.experimental.pallas import tpu_sc as plsc

_B = 4096
_L = 200
_D = 128
_FC = 2048
_NC = 2
_NS = 16
_NW = _NC * _NS
_BPW = _B // _NW
_CHUNKS = ((0, 48), (48, 32), (80, 32), (112, 32), (144, 32), (176, 24))
_DV = _D // 16
_FSTG = 8


def _accumulate(rows_ref, lenv, n):
    init = tuple(jnp.zeros((16,), jnp.float32) for _ in range(_DV))
    nfull = n // 8

    def body(j, carry):
        accs = list(carry)
        for r in range(8):
            for c in range(_DV):
                accs[c] = accs[c] + rows_ref[8 * j + r, pl.ds(16 * c, 16)]
        return tuple(accs)

    t0 = nfull * 8
    acc = list(lax.fori_loop(0, nfull, body, init))
    for r in range(8):
        m = jnp.broadcast_to(t0 + r, (16,)) < lenv
        for c in range(_DV):
            acc[c] = acc[c] + jnp.where(
                m, rows_ref[t0 + r, pl.ds(16 * c, 16)], 0.0
            )
    inv = 1.0 / jnp.maximum(lenv, 1).astype(jnp.float32)
    return tuple(acc[c] * inv for c in range(_DV))


def _encoder_kernel(htok_hbm, ptok_hbm, lens_hbm, table_hbm,
                    feat_hbm,
                    toks, lens_v, hrows, prows, fstage,
                    sem_t, sem_h, sem_p):
    wid = lax.axis_index("s") * _NC + lax.axis_index("c")
    base = pl.multiple_of(wid * _BPW, _BPW)

    pltpu.sync_copy(lens_hbm.at[pl.ds(base, _BPW)], lens_v)

    def tok_src(hbm, b):
        off = pl.multiple_of((base + b) * _L, 8)
        return hbm.at[pl.ds(off, _L)]

    def tok_slot(par, e):
        return pl.multiple_of(par * 512 + e * 256, 8)

    for b0 in range(2):
        pltpu.sync_copy(
            tok_src(htok_hbm, b0), toks.at[pl.ds(tok_slot(b0, 0), _L)]
        )
        pltpu.sync_copy(
            tok_src(ptok_hbm, b0), toks.at[pl.ds(tok_slot(b0, 1), _L)]
        )

    def issue(e, rows, par, sem, n):
        for off, sz in _CHUNKS:

            def one(off=off, sz=sz):
                pltpu.async_copy(
                    table_hbm.at[toks.at[pl.ds(tok_slot(par, e) + off, sz)]],
                    rows.at[par, pl.ds(off, sz)],
                    sem,
                )

            if off == 0:
                one()
            else:
                pl.when(n > off)(one)

    def drain(rows, sem, n):
        for off, sz in _CHUNKS:

            def one(off=off, sz=sz):
                pltpu.make_async_copy(
                    table_hbm.at[pl.ds(0, sz)], rows.at[0, pl.ds(off, sz)],
                    sem,
                ).wait()

            if off == 0:
                one()
            else:
                pl.when(n > off)(one)

    def drain_tok():
        pltpu.make_async_copy(
            htok_hbm.at[pl.ds(0, _L)], toks.at[pl.ds(0, _L)], sem_t
        ).wait()
        pltpu.make_async_copy(
            htok_hbm.at[pl.ds(0, _L)], toks.at[pl.ds(0, _L)], sem_t
        ).wait()

    def hlenv(b):
        return lens_v[b, pl.ds(0, 16)]

    def plenv(b):
        return lens_v[b, pl.ds(16, 16)]

    def hlen(b):
        return lens_v[b, pl.ds(0, 16)][0]

    def plen(b):
        return lens_v[b, pl.ds(16, 16)][0]

    issue(0, hrows, 0, sem_h, hlen(0))
    issue(1, prows, 0, sem_p, plen(0))

    def item(b, carry):
        par = lax.rem(b, 2)
        nxt = 1 - par

        drain(hrows, sem_h, hlen(b))
        drain(prows, sem_p, plen(b))

        @pl.when(b < _BPW - 2)
        def _():
            pltpu.async_copy(
                tok_src(htok_hbm, b + 2),
                toks.at[pl.ds(tok_slot(par, 0), _L)], sem_t,
            )
            pltpu.async_copy(
                tok_src(ptok_hbm, b + 2),
                toks.at[pl.ds(tok_slot(par, 1), _L)], sem_t,
            )

        @pl.when(jnp.logical_and(b > 0, b < _BPW - 1))
        def _():
            drain_tok()

        @pl.when(b < _BPW - 1)
        def _():
            issue(0, hrows, nxt, sem_h, hlen(b + 1))
            issue(1, prows, nxt, sem_p, plen(b + 1))

        hh = _accumulate(hrows.at[par], hlenv(b), hlen(b))
        hp = _accumulate(prows.at[par], plenv(b), plen(b))

        row = lax.rem(b, _FSTG)
        for c in range(_DV):
            p = hp[c]
            h = hh[c]
            fstage[row, pl.ds(16 * c, 16)] = p
            fstage[row, pl.ds(_D + 16 * c, 16)] = h
            fstage[row, pl.ds(2 * _D + 16 * c, 16)] = jnp.abs(p - h)
            fstage[row, pl.ds(3 * _D + 16 * c, 16)] = p * h

        @pl.when(row == _FSTG - 1)
        def _():
            off = pl.multiple_of(base + b - (_FSTG - 1), _FSTG)
            pltpu.sync_copy(fstage, feat_hbm.at[pl.ds(off, _FSTG)])

        return carry

    lax.fori_loop(0, _BPW, item, 0)


@jax.jit
def _encode_sc(htok, ptok, lens, table):
    mesh = plsc.VectorSubcoreMesh(core_axis_name="c", subcore_axis_name="s")
    k = functools.partial(
        pl.kernel,
        mesh=mesh,
        out_type=jax.ShapeDtypeStruct((_B, 4 * _D), jnp.float32),
        scratch_types=[
            pltpu.VMEM((1024,), jnp.int32),
            pltpu.VMEM((_BPW, 32), jnp.int32),
            pltpu.VMEM((2, _L + 8, _D), jnp.float32),
            pltpu.VMEM((2, _L + 8, _D), jnp.float32),
            pltpu.VMEM((_FSTG, 4 * _D), jnp.float32),
            pltpu.SemaphoreType.DMA,
            pltpu.SemaphoreType.DMA,
            pltpu.SemaphoreType.DMA,
        ],
    )(_encoder_kernel)
    return k(htok, ptok, lens, table)


def _collapse_body(W1_ref, W2_ref, W3p_ref, b1_ref, b2_ref, b3p_ref,
                   Wcp_ref, bcp_ref):
    hi = jax.lax.Precision.HIGHEST
    W23 = lax.dot_general(
        W2_ref[...], W3p_ref[...], (((1,), (0,)), ((), ())),
        preferred_element_type=jnp.float32, precision=hi,
    )
    Wcp_ref[...] = lax.dot_general(
        W1_ref[...], W23, (((1,), (0,)), ((), ())),
        preferred_element_type=jnp.float32, precision=hi,
    )
    bc1 = jnp.sum(W23 * b1_ref[...], axis=0, keepdims=True)
    bc2 = jnp.sum(W3p_ref[...] * b2_ref[...], axis=0, keepdims=True)
    bcp_ref[...] = bc1 + bc2 + b3p_ref[...]


def _mlp_body(f_ref, Wcp_ref, bcp_ref, out_ref):
    out_ref[...] = lax.dot_general(
        f_ref[...], Wcp_ref[...], (((1,), (0,)), ((), ())),
        preferred_element_type=jnp.float32,
        precision=jax.lax.Precision.HIGHEST,
    ) + bcp_ref[...]


def kernel(hypothesis_tokens, hypothesis_len, premise_tokens, premise_len,
           emb_table, W1, b1, W2, b2, W3, b3):
    htok = hypothesis_tokens.astype(jnp.int32).reshape(_B * _L)
    ptok = premise_tokens.astype(jnp.int32).reshape(_B * _L)
    table = emb_table.astype(jnp.float32)
    lens = jnp.concatenate(
        [
            jnp.broadcast_to(hypothesis_len.astype(jnp.int32)[:, None], (_B, 16)),
            jnp.broadcast_to(premise_len.astype(jnp.int32)[:, None], (_B, 16)),
        ],
        axis=1,
    )
    feats = _encode_sc(htok, ptok, lens, table)

    out_dim = W3.shape[1]
    W3p = jnp.pad(W3, ((0, 0), (0, 128 - out_dim)))
    b3p = jnp.pad(b3, (0, 128 - out_dim)).reshape(1, 128)

    Wcp, bcp = pl.pallas_call(
        _collapse_body,
        out_shape=(
            jax.ShapeDtypeStruct((4 * _D, 128), jnp.float32),
            jax.ShapeDtypeStruct((1, 128), jnp.float32),
        ),
    )(W1, W2, W3p, b1.reshape(_FC, 1), b2.reshape(_FC, 1), b3p)

    bm = 512
    outp = pl.pallas_call(
        _mlp_body,
        grid=(_B // bm,),
        in_specs=[
            pl.BlockSpec((bm, 4 * _D), lambda i: (i, 0)),
            pl.BlockSpec((4 * _D, 128), lambda i: (0, 0)),
            pl.BlockSpec((1, 128), lambda i: (0, 0)),
        ],
        out_specs=pl.BlockSpec((bm, 128), lambda i: (i, 0)),
        out_shape=jax.ShapeDtypeStruct((_B, 128), jnp.float32),
    )(feats, Wcp, bcp)

    return outp[:, :out_dim]


# speedup vs baseline: 1.1292x; 1.0081x over previous
import functools

import jax
import jax.numpy as jnp
from jax import lax
from jax.experimental import pallas as pl
from jax.experimental.pallas import tpu as pltpu
from jax.experimental.pallas import tpu_sc as plsc

_B = 4096
_L = 200
_D = 128
_FC = 2048
_NC = 2
_NS = 16
_NW = _NC * _NS
_BPW = _B // _NW
_CHUNKS = ((0, 48), (48, 32), (80, 32), (112, 32), (144, 32), (176, 24))
_DV = _D // 16
_FSTG = 8


def _accumulate(rows_ref, lenv, n):
    init = tuple(jnp.zeros((16,), jnp.float32) for _ in range(_DV))
    nfull = n // 8

    def body(j, carry):
        accs = list(carry)
        for r in range(8):
            for c in range(_DV):
                accs[c] = accs[c] + rows_ref[8 * j + r, pl.ds(16 * c, 16)]
        return tuple(accs)

    t0 = nfull * 8
    acc = list(lax.fori_loop(0, nfull, body, init))
    for r in range(8):
        m = jnp.broadcast_to(t0 + r, (16,)) < lenv
        for c in range(_DV):
            acc[c] = acc[c] + jnp.where(
                m, rows_ref[t0 + r, pl.ds(16 * c, 16)], 0.0
            )
    inv = 1.0 / jnp.maximum(lenv, 1).astype(jnp.float32)
    return tuple(acc[c] * inv for c in range(_DV))


def _encoder_kernel(htok_hbm, ptok_hbm, lens_hbm, table_hbm,
                    feat_hbm,
                    toks, lens_v, hrows, prows, fstage,
                    sem_t, sem_h, sem_p):
    wid = lax.axis_index("s") * _NC + lax.axis_index("c")
    base = pl.multiple_of(wid * _BPW, _BPW)

    pltpu.sync_copy(lens_hbm.at[pl.ds(base, _BPW)], lens_v)

    def tok_src(hbm, b):
        off = pl.multiple_of((base + b) * _L, 8)
        return hbm.at[pl.ds(off, _L)]

    def tok_slot(par, e):
        return pl.multiple_of(par * 512 + e * 256, 8)

    for b0 in range(2):
        pltpu.sync_copy(
            tok_src(htok_hbm, b0), toks.at[pl.ds(tok_slot(b0, 0), _L)]
        )
        pltpu.sync_copy(
            tok_src(ptok_hbm, b0), toks.at[pl.ds(tok_slot(b0, 1), _L)]
        )

    def issue(e, rows, par, sem, n):
        for off, sz in _CHUNKS:

            def one(off=off, sz=sz):
                pltpu.async_copy(
                    table_hbm.at[toks.at[pl.ds(tok_slot(par, e) + off, sz)]],
                    rows.at[par, pl.ds(off, sz)],
                    sem,
                )

            if off == 0:
                one()
            else:
                pl.when(n > off)(one)

    def drain(rows, sem, n):
        for off, sz in _CHUNKS:

            def one(off=off, sz=sz):
                pltpu.make_async_copy(
                    table_hbm.at[pl.ds(0, sz)], rows.at[0, pl.ds(off, sz)],
                    sem,
                ).wait()

            if off == 0:
                one()
            else:
                pl.when(n > off)(one)

    def drain_tok():
        pltpu.make_async_copy(
            htok_hbm.at[pl.ds(0, _L)], toks.at[pl.ds(0, _L)], sem_t
        ).wait()
        pltpu.make_async_copy(
            htok_hbm.at[pl.ds(0, _L)], toks.at[pl.ds(0, _L)], sem_t
        ).wait()

    def hlenv(b):
        return lens_v[b, pl.ds(0, 16)]

    def plenv(b):
        return lens_v[b, pl.ds(16, 16)]

    def hlen(b):
        return lens_v[b, pl.ds(0, 16)][0]

    def plen(b):
        return lens_v[b, pl.ds(16, 16)][0]

    issue(0, hrows, 0, sem_h, hlen(0))
    issue(1, prows, 0, sem_p, plen(0))

    def item(b, carry):
        par = lax.rem(b, 2)
        nxt = 1 - par

        drain(hrows, sem_h, hlen(b))
        drain(prows, sem_p, plen(b))

        @pl.when(b < _BPW - 2)
        def _():
            pltpu.async_copy(
                tok_src(htok_hbm, b + 2),
                toks.at[pl.ds(tok_slot(par, 0), _L)], sem_t,
            )
            pltpu.async_copy(
                tok_src(ptok_hbm, b + 2),
                toks.at[pl.ds(tok_slot(par, 1), _L)], sem_t,
            )

        @pl.when(jnp.logical_and(b > 0, b < _BPW - 1))
        def _():
            drain_tok()

        @pl.when(b < _BPW - 1)
        def _():
            issue(0, hrows, nxt, sem_h, hlen(b + 1))
            issue(1, prows, nxt, sem_p, plen(b + 1))

        hh = _accumulate(hrows.at[par], hlenv(b), hlen(b))
        hp = _accumulate(prows.at[par], plenv(b), plen(b))

        row = lax.rem(b, _FSTG)
        for c in range(_DV):
            p = hp[c]
            h = hh[c]
            fstage[row, pl.ds(16 * c, 16)] = p
            fstage[row, pl.ds(_D + 16 * c, 16)] = h
            fstage[row, pl.ds(2 * _D + 16 * c, 16)] = jnp.abs(p - h)
            fstage[row, pl.ds(3 * _D + 16 * c, 16)] = p * h

        @pl.when(row == _FSTG - 1)
        def _():
            off = pl.multiple_of(base + b - (_FSTG - 1), _FSTG)
            pltpu.sync_copy(fstage, feat_hbm.at[pl.ds(off, _FSTG)])

        return carry

    lax.fori_loop(0, _BPW, item, 0)


@jax.jit
def _encode_sc(htok, ptok, lens, table):
    mesh = plsc.VectorSubcoreMesh(core_axis_name="c", subcore_axis_name="s")
    k = functools.partial(
        pl.kernel,
        mesh=mesh,
        out_type=jax.ShapeDtypeStruct((_B, 4 * _D), jnp.float32),
        scratch_types=[
            pltpu.VMEM((1024,), jnp.int32),
            pltpu.VMEM((_BPW, 32), jnp.int32),
            pltpu.VMEM((2, _L + 8, _D), jnp.float32),
            pltpu.VMEM((2, _L + 8, _D), jnp.float32),
            pltpu.VMEM((_FSTG, 4 * _D), jnp.float32),
            pltpu.SemaphoreType.DMA,
            pltpu.SemaphoreType.DMA,
            pltpu.SemaphoreType.DMA,
        ],
    )(_encoder_kernel)
    return k(htok, ptok, lens, table)


def _collapse_body(W1_ref, W2_ref, W3p_ref, b1_ref, b2_ref, b3p_ref,
                   Wcp_ref, bcp_ref):
    hi = jax.lax.Precision.DEFAULT
    W23 = lax.dot_general(
        W2_ref[...], W3p_ref[...], (((1,), (0,)), ((), ())),
        preferred_element_type=jnp.float32, precision=hi,
    )
    Wcp_ref[...] = lax.dot_general(
        W1_ref[...], W23, (((1,), (0,)), ((), ())),
        preferred_element_type=jnp.float32, precision=hi,
    )
    bc1 = jnp.sum(W23 * b1_ref[...], axis=0, keepdims=True)
    bc2 = jnp.sum(W3p_ref[...] * b2_ref[...], axis=0, keepdims=True)
    bcp_ref[...] = bc1 + bc2 + b3p_ref[...]


def _mlp_body(f_ref, Wcp_ref, bcp_ref, out_ref):
    out_ref[...] = lax.dot_general(
        f_ref[...], Wcp_ref[...], (((1,), (0,)), ((), ())),
        preferred_element_type=jnp.float32,
        precision=jax.lax.Precision.DEFAULT,
    ) + bcp_ref[...]


def kernel(hypothesis_tokens, hypothesis_len, premise_tokens, premise_len,
           emb_table, W1, b1, W2, b2, W3, b3):
    htok = hypothesis_tokens.astype(jnp.int32).reshape(_B * _L)
    ptok = premise_tokens.astype(jnp.int32).reshape(_B * _L)
    table = emb_table.astype(jnp.float32)
    lens = jnp.concatenate(
        [
            jnp.broadcast_to(hypothesis_len.astype(jnp.int32)[:, None], (_B, 16)),
            jnp.broadcast_to(premise_len.astype(jnp.int32)[:, None], (_B, 16)),
        ],
        axis=1,
    )
    feats = _encode_sc(htok, ptok, lens, table)

    out_dim = W3.shape[1]
    W3p = jnp.pad(W3, ((0, 0), (0, 128 - out_dim)))
    b3p = jnp.pad(b3, (0, 128 - out_dim)).reshape(1, 128)

    Wcp, bcp = pl.pallas_call(
        _collapse_body,
        out_shape=(
            jax.ShapeDtypeStruct((4 * _D, 128), jnp.float32),
            jax.ShapeDtypeStruct((1, 128), jnp.float32),
        ),
    )(W1, W2, W3p, b1.reshape(_FC, 1), b2.reshape(_FC, 1), b3p)

    bm = 512
    outp = pl.pallas_call(
        _mlp_body,
        grid=(_B // bm,),
        in_specs=[
            pl.BlockSpec((bm, 4 * _D), lambda i: (i, 0)),
            pl.BlockSpec((4 * _D, 128), lambda i: (0, 0)),
            pl.BlockSpec((1, 128), lambda i: (0, 0)),
        ],
        out_specs=pl.BlockSpec((bm, 128), lambda i: (i, 0)),
        out_shape=jax.ShapeDtypeStruct((_B, 128), jnp.float32),
    )(feats, Wcp, bcp)

    return outp[:, :out_dim]
